# baseline (speedup 1.0000x reference)
import functools

import jax
import jax.numpy as jnp
from jax import lax
from jax.experimental import pallas as pl
from jax.experimental.pallas import tpu as pltpu
from jax.experimental.pallas import tpu_sc as plsc

_ROUNDS = 4
_NB2 = 32
_CHUNK = 512
_L = 16


def _tc_body(inp_ref, rm_ref, m_ref):
    rmT = rm_ref[0]
    rmn = rmT / jnp.sqrt(jnp.sum(rmT * rmT, axis=1, keepdims=True))
    x = inp_ref[0]
    ss = jnp.sum(x * x, axis=1)
    nrm = jnp.maximum(jnp.sqrt(ss), 1e-12)
    xn = x / nrm[:, None]
    m_ref[0] = jax.lax.dot_general(
        rmn.astype(jnp.bfloat16), xn.astype(jnp.bfloat16),
        dimension_numbers=(((1,), (1,)), ((), ())),
        preferred_element_type=jnp.float32)


def _sc_body(m_hbm, out_hbm, m_v, o_v, *, batch, length):
    nw = 32
    wid = lax.axis_index("s") * 2 + lax.axis_index("c")
    bpw = batch // nw

    def chunk_body(ci, _):
        b = wid * bpw + ci // (length // _CHUNK)
        c = ci % (length // _CHUNK)
        pltpu.sync_copy(m_hbm.at[b, :, pl.ds(c * _CHUNK, _CHUNK)], m_v)
        tok0 = c * _CHUNK

        def group_body(g, _):
            sl = pl.ds(g * _L, _L)
            for r in range(_ROUNDS):
                v = m_v[r * _NB2, sl]
                a = jnp.abs(v)
                amax = a
                idx = jnp.where(v < 0, jnp.full((_L,), _NB2, jnp.int32),
                                jnp.zeros((_L,), jnp.int32))
                for j in range(1, _NB2):
                    v = m_v[r * _NB2 + j, sl]
                    a = jnp.abs(v)
                    kj = jnp.where(v < 0,
                                   jnp.full((_L,), j + _NB2, jnp.int32),
                                   jnp.full((_L,), j, jnp.int32))
                    gt = a > amax
                    upd = gt | ((a == amax) & (kj < idx))
                    amax = jnp.where(gt, a, amax)
                    idx = jnp.where(upd, kj, idx)
                tok = (lax.iota(jnp.int32, _L) + (tok0 + g * _L))
                o_v[r, sl] = idx * length + tok
            return 0

        lax.fori_loop(0, _CHUNK // _L, group_body, 0)
        pltpu.sync_copy(o_v, out_hbm.at[b, :, pl.ds(c * _CHUNK, _CHUNK)])
        return 0

    lax.fori_loop(0, bpw * (length // _CHUNK), chunk_body, 0)


def kernel(inp, rand_matrix, n_buckets):
    del n_buckets
    batch, length, d_k = inp.shape
    rounds, nb2 = rand_matrix.shape[2], rand_matrix.shape[3]
    rmT = rand_matrix.transpose(0, 2, 3, 1).reshape(batch, rounds * nb2, d_k)

    mT = pl.pallas_call(
        _tc_body,
        grid=(batch,),
        in_specs=[
            pl.BlockSpec((1, length, d_k), lambda b: (b, 0, 0)),
            pl.BlockSpec((1, rounds * nb2, d_k), lambda b: (b, 0, 0)),
        ],
        out_specs=pl.BlockSpec((1, rounds * nb2, length), lambda b: (b, 0, 0)),
        out_shape=jax.ShapeDtypeStruct((batch, rounds * nb2, length),
                                       jnp.float32),
        compiler_params=pltpu.CompilerParams(
            dimension_semantics=("arbitrary",),
        ),
    )(inp, rmT)

    sc = functools.partial(
        pl.kernel,
        out_type=jax.ShapeDtypeStruct((batch, rounds, length), jnp.int32),
        mesh=plsc.VectorSubcoreMesh(core_axis_name="c", subcore_axis_name="s"),
        scratch_types=[
            pltpu.VMEM((rounds * nb2, _CHUNK), jnp.float32),
            pltpu.VMEM((rounds, _CHUNK), jnp.int32),
        ],
    )(functools.partial(_sc_body, batch=batch, length=length))
    out = sc(mT)
    return out.swapaxes(1, 2)


# speedup vs baseline: 1.9113x; 1.9113x over previous
import functools

import jax
import jax.numpy as jnp
from jax.experimental import pallas as pl
from jax.experimental.pallas import tpu as pltpu

_ROUNDS = 4
_NB2 = 32


def _lsh_body(inp_ref, rm_ref, out_ref, *, length):
    rmT = rm_ref[0]
    rmn = rmT / jnp.sqrt(jnp.sum(rmT * rmT, axis=1, keepdims=True))

    x = inp_ref[0]
    ss = jnp.sum(x * x, axis=1)
    nrm = jnp.maximum(jnp.sqrt(ss), 1e-12)
    xn = x / nrm[:, None]
    xb = xn.astype(jnp.bfloat16)
    mT = jax.lax.dot_general(
        rmn.astype(jnp.bfloat16), xb,
        dimension_numbers=(((1,), (1,)), ((), ())),
        preferred_element_type=jnp.float32)

    av = jnp.abs(mT)
    rows = jax.lax.broadcasted_iota(jnp.int32, (_ROUNDS * _NB2, length), 0)
    key_all = (rows % _NB2) + jnp.where(mT < 0, _NB2, 0)
    tok = jax.lax.broadcasted_iota(jnp.int32, (1, length), 1)
    cols = []
    for r in range(_ROUNDS):
        ar = av[r * _NB2:(r + 1) * _NB2]
        amax = jnp.max(ar, axis=0, keepdims=True)
        key = jnp.where(ar == amax, key_all[r * _NB2:(r + 1) * _NB2],
                        2 * _NB2)
        h = jnp.min(key, axis=0, keepdims=True)
        cols.append(h * length + tok)
    out_ref[0] = jnp.concatenate(cols, axis=0)


def kernel(inp, rand_matrix, n_buckets):
    del n_buckets
    batch, length, d_k = inp.shape
    rounds, nb2 = rand_matrix.shape[2], rand_matrix.shape[3]
    rmT = rand_matrix.transpose(0, 2, 3, 1).reshape(batch, rounds * nb2, d_k)
    out = pl.pallas_call(
        functools.partial(_lsh_body, length=length),
        grid=(batch,),
        in_specs=[
            pl.BlockSpec((1, length, d_k), lambda b: (b, 0, 0)),
            pl.BlockSpec((1, rounds * nb2, d_k), lambda b: (b, 0, 0)),
        ],
        out_specs=pl.BlockSpec((1, rounds, length), lambda b: (b, 0, 0)),
        out_shape=jax.ShapeDtypeStruct((batch, rounds, length), jnp.int32),
        compiler_params=pltpu.CompilerParams(
            dimension_semantics=("arbitrary",),
        ),
    )(inp, rmT)
    return out.swapaxes(1, 2)
